# Initial kernel scaffold; baseline (speedup 1.0000x reference)
import jax
import jax.numpy as jnp
from jax.experimental import pallas as pl


def kernel(queries, keys, values):
    raise NotImplementedError("write your pallas kernel here")


# speedup vs baseline: 1.4318x; 1.4318x over previous
import math

import jax
import jax.numpy as jnp
import numpy as np
from jax.experimental import pallas as pl
from jax.experimental.pallas import tpu as pltpu

_L = 2048
_D = 64
_U = 40
_CHUNK = 256
_NEG = -1e30


def _build_counts() -> np.ndarray:
    idx = np.asarray(
        jax.random.randint(jax.random.key(1234), (_L, _U), 0, _L)
    )
    c = np.zeros((_L, _L), np.int8)
    np.add.at(c, (np.arange(_L)[:, None], idx), 1)
    return c


_COUNTS = _build_counts()


def _body(c_ref, q_ref, k_ref, v_ref, o_ref):
    q = q_ref[0]
    k = k_ref[0]
    v = v_ref[0]

    m_parts = []
    for i in range(_L // _CHUNK):
        qc = q[i * _CHUNK:(i + 1) * _CHUNK, :]
        sc = jax.lax.dot_general(
            qc, k, (((1,), (1,)), ((), ())),
            preferred_element_type=jnp.float32,
            precision=jax.lax.Precision.HIGHEST,
        )
        cc = c_ref[i * _CHUNK:(i + 1) * _CHUNK, :].astype(jnp.float32)
        smax = jnp.max(jnp.where(cc > 0.0, sc, _NEG), axis=1, keepdims=True)
        ssum = jnp.sum(sc * cc, axis=1, keepdims=True)
        m_parts.append(smax - ssum / jnp.float32(_L))
    m = jnp.concatenate(m_parts, axis=0)

    sub_iota = jax.lax.broadcasted_iota(jnp.int32, (_L, 1), 0)
    lane_iota = jax.lax.broadcasted_iota(jnp.int32, (1, _L), 1)
    big = jnp.int32(_L)
    oh_rows = []
    for _ in range(_U):
        gmax = jnp.max(m)
        at_max = m >= gmax
        idx = jnp.min(jnp.where(at_max, sub_iota, big))
        oh_rows.append((lane_iota == idx).astype(jnp.float32))
        m = jnp.where(sub_iota == idx, _NEG, m)
    oh = jnp.concatenate(oh_rows, axis=0)

    q_r = jnp.dot(oh, q, preferred_element_type=jnp.float32,
                  precision=jax.lax.Precision.HIGHEST)
    scores = jax.lax.dot_general(
        q_r, k, (((1,), (1,)), ((), ())),
        preferred_element_type=jnp.float32,
        precision=jax.lax.Precision.HIGHEST,
    ) * jnp.float32(1.0 / math.sqrt(_D))
    smax = jnp.max(scores, axis=1, keepdims=True)
    e = jnp.exp(scores - smax)
    p = e / jnp.sum(e, axis=1, keepdims=True)
    upd = jnp.dot(p, v, preferred_element_type=jnp.float32,
                  precision=jax.lax.Precision.HIGHEST)

    vmean = jnp.mean(v, axis=0, keepdims=True)
    delta = upd - vmean
    ctx = vmean + jax.lax.dot_general(
        oh, delta, (((0,), (0,)), ((), ())),
        preferred_element_type=jnp.float32,
        precision=jax.lax.Precision.HIGHEST,
    )
    o_ref[0] = ctx


def kernel(queries, keys, values):
    b = queries.shape[0]
    counts = jnp.asarray(_COUNTS)
    context = pl.pallas_call(
        _body,
        grid=(b,),
        in_specs=[
            pl.BlockSpec((_L, _L), lambda i: (0, 0)),
            pl.BlockSpec((1, _L, _D), lambda i: (i, 0, 0)),
            pl.BlockSpec((1, _L, _D), lambda i: (i, 0, 0)),
            pl.BlockSpec((1, _L, _D), lambda i: (i, 0, 0)),
        ],
        out_specs=pl.BlockSpec((1, _L, _D), lambda i: (i, 0, 0)),
        out_shape=jax.ShapeDtypeStruct((b, _L, _D), jnp.float32),
    )(counts, queries, keys, values)
    return (context, None)


# speedup vs baseline: 2.1148x; 1.4771x over previous
import math

import jax
import jax.numpy as jnp
import numpy as np
from jax.experimental import pallas as pl
from jax.experimental.pallas import tpu as pltpu

_L = 2048
_D = 64
_U = 40
_CHUNK = 256
_NEG = -1e30
_HI = jax.lax.Precision.HIGHEST


def _build_tables():
    idx = np.asarray(
        jax.random.randint(jax.random.key(1234), (_L, _U), 0, _L)
    )
    c = np.zeros((_L, _L), np.float32)
    np.add.at(c, (np.arange(_L)[:, None], idx), 1.0)
    ct = np.ascontiguousarray(c.T)
    bias_t = np.where(ct > 0.0, 0.0, np.float32(_NEG)).astype(np.float32)
    return ct, bias_t


_COUNTS_T, _BIAS_T = _build_tables()


def _body(ct_ref, bt_ref, q_ref, k_ref, v_ref, o_ref):
    q = q_ref[0]
    k = k_ref[0]
    v = v_ref[0]

    m_parts = []
    for i in range(_L // _CHUNK):
        qc = q[i * _CHUNK:(i + 1) * _CHUNK, :]
        st = jax.lax.dot_general(
            k, qc, (((1,), (1,)), ((), ())),
            preferred_element_type=jnp.float32, precision=_HI,
        )
        bt = bt_ref[:, i * _CHUNK:(i + 1) * _CHUNK]
        ct = ct_ref[:, i * _CHUNK:(i + 1) * _CHUNK]
        smax = jnp.max(st + bt, axis=0, keepdims=True)
        ssum = jnp.sum(st * ct, axis=0, keepdims=True)
        m_parts.append(smax - ssum * jnp.float32(1.0 / _L))
    m = jnp.concatenate(m_parts, axis=1)

    lane_iota = jax.lax.broadcasted_iota(jnp.int32, (1, _L), 1)
    oh_rows = []
    for _ in range(_U):
        gmax = jnp.max(m)
        idx = jnp.min(jnp.where(m >= gmax, lane_iota, jnp.int32(_L)))
        hit = lane_iota == idx
        oh_rows.append(hit.astype(jnp.float32))
        m = jnp.where(hit, jnp.float32(_NEG), m)
    oh = jnp.concatenate(oh_rows, axis=0)

    q_r = jnp.dot(oh, q, preferred_element_type=jnp.float32,
                  precision=_HI)
    scores = jax.lax.dot_general(
        q_r, k, (((1,), (1,)), ((), ())),
        preferred_element_type=jnp.float32, precision=_HI,
    ) * jnp.float32(1.0 / math.sqrt(_D))
    smax = jnp.max(scores, axis=1, keepdims=True)
    e = jnp.exp(scores - smax)
    p = e / jnp.sum(e, axis=1, keepdims=True)
    upd = jnp.dot(p, v, preferred_element_type=jnp.float32,
                  precision=_HI)

    vmean = jnp.mean(v, axis=0, keepdims=True)
    delta = upd - vmean
    ctx = vmean + jax.lax.dot_general(
        oh, delta, (((0,), (0,)), ((), ())),
        preferred_element_type=jnp.float32, precision=_HI,
    )
    o_ref[0] = ctx


def kernel(queries, keys, values):
    b = queries.shape[0]
    counts_t = jnp.asarray(_COUNTS_T)
    bias_t = jnp.asarray(_BIAS_T)
    context = pl.pallas_call(
        _body,
        grid=(b,),
        in_specs=[
            pl.BlockSpec((_L, _L), lambda i: (0, 0)),
            pl.BlockSpec((_L, _L), lambda i: (0, 0)),
            pl.BlockSpec((1, _L, _D), lambda i: (i, 0, 0)),
            pl.BlockSpec((1, _L, _D), lambda i: (i, 0, 0)),
            pl.BlockSpec((1, _L, _D), lambda i: (i, 0, 0)),
        ],
        out_specs=pl.BlockSpec((1, _L, _D), lambda i: (i, 0, 0)),
        out_shape=jax.ShapeDtypeStruct((b, _L, _D), jnp.float32),
    )(counts_t, bias_t, queries, keys, values)
    return (context, None)


# speedup vs baseline: 4.1034x; 1.9403x over previous
import math

import jax
import jax.numpy as jnp
import numpy as np
from jax.experimental import pallas as pl

_L = 2048
_D = 64
_U = 40
_CHUNK = 256
_BB = 4
_NEG = -1e30


def _threefry2x32(k1, k2, x1, x2):
    rot = ([13, 15, 26, 6], [17, 29, 16, 24])
    k1 = np.uint32(k1)
    k2 = np.uint32(k2)
    ks = [k1, k2, np.uint32(k1 ^ k2 ^ np.uint32(0x1BD11BDA))]
    x = [x1.astype(np.uint32) + ks[0], x2.astype(np.uint32) + ks[1]]
    for i in range(5):
        for r in rot[i % 2]:
            x[0] = x[0] + x[1]
            x[1] = (x[1] << np.uint32(r)) | (x[1] >> np.uint32(32 - r))
            x[1] = x[0] ^ x[1]
        x[0] = x[0] + ks[(i + 1) % 3]
        x[1] = x[1] + ks[(i + 2) % 3] + np.uint32(i + 1)
    return x


def _sample_indices():
    size = _L * _U
    b1, b2 = _threefry2x32(
        0, 1234, np.zeros(2, np.uint32), np.arange(2, dtype=np.uint32)
    )
    h1, h2 = _threefry2x32(
        b1[1], b2[1],
        np.zeros(size, np.uint32), np.arange(size, dtype=np.uint32),
    )
    bits = (h1 ^ h2).reshape(_L, _U)
    return (bits % np.uint32(_L)).astype(np.int32)


def _build_tables():
    idx = _sample_indices()
    c = np.zeros((_L, _L), np.int8)
    np.add.at(c, (np.arange(_L)[:, None], idx), 1)
    return np.ascontiguousarray(c.T)


_COUNTS_T = _build_tables()


def _split3(x):
    x0 = x.astype(jnp.bfloat16)
    r1 = x - x0.astype(jnp.float32)
    x1 = r1.astype(jnp.bfloat16)
    x2 = (r1 - x1.astype(jnp.float32)).astype(jnp.bfloat16)
    return x0, x1, x2


def _body(ct_ref, q_ref, k_ref, v_ref, o_ref):
    qcats, kcats, vs = [], [], []
    for j in range(_BB):
        q0, q1, q2 = _split3(q_ref[j])
        k0, k1, k2 = _split3(k_ref[j])
        qcats.append(jnp.concatenate([q0, q0, q1, q0, q2, q1], axis=1))
        kcats.append(jnp.concatenate([k0, k1, k0, k2, k0, k1], axis=1))
        vs.append(v_ref[j])

    m_parts = [[] for _ in range(_BB)]
    for i in range(_L // _CHUNK):
        ct8 = ct_ref[:, i * _CHUNK:(i + 1) * _CHUNK]
        ct = ct8.astype(jnp.float32)
        mask = ct > 0.0
        for j in range(_BB):
            qcc = qcats[j][i * _CHUNK:(i + 1) * _CHUNK, :]
            st = jax.lax.dot_general(
                kcats[j], qcc, (((1,), (1,)), ((), ())),
                preferred_element_type=jnp.float32,
            )
            smax = jnp.max(jnp.where(mask, st, jnp.float32(_NEG)),
                           axis=0, keepdims=True)
            ssum = jnp.sum(st * ct, axis=0, keepdims=True)
            m_parts[j].append(smax - ssum * jnp.float32(1.0 / _L))
    ms = [jnp.concatenate(p, axis=1) for p in m_parts]

    lane_iota = jax.lax.broadcasted_iota(jnp.int32, (1, _L), 1)
    oh_rows = [[] for _ in range(_BB)]
    for _ in range(_U):
        for j in range(_BB):
            gmax = jnp.max(ms[j])
            idx = jnp.min(jnp.where(ms[j] >= gmax, lane_iota, jnp.int32(_L)))
            hit = lane_iota == idx
            oh_rows[j].append(hit.astype(jnp.bfloat16))
            ms[j] = jnp.where(hit, jnp.float32(_NEG), ms[j])
    ohs = [jnp.concatenate(r, axis=0) for r in oh_rows]

    sc = []
    for j in range(_BB):
        q_r_cat = jax.lax.dot_general(
            ohs[j], qcats[j], (((1,), (0,)), ((), ())),
            preferred_element_type=jnp.float32,
        ).astype(jnp.bfloat16)
        sc.append(jax.lax.dot_general(
            q_r_cat, kcats[j], (((1,), (1,)), ((), ())),
            preferred_element_type=jnp.float32,
        ) * jnp.float32(1.0 / math.sqrt(_D)))
    scores = jnp.concatenate(sc, axis=0)
    smax = jnp.max(scores, axis=1, keepdims=True)
    e = jnp.exp(scores - smax)
    p = e / jnp.sum(e, axis=1, keepdims=True)
    p16 = p.astype(jnp.bfloat16)

    for j in range(_BB):
        v = vs[j]
        v0 = v.astype(jnp.bfloat16)
        v1 = (v - v0.astype(jnp.float32)).astype(jnp.bfloat16)
        upd = jax.lax.dot_general(
            p16[j * _U:(j + 1) * _U], jnp.concatenate([v0, v1], axis=1),
            (((1,), (0,)), ((), ())),
            preferred_element_type=jnp.float32,
        )
        upd = upd[:, :_D] + upd[:, _D:]
        vmean = jnp.mean(v, axis=0, keepdims=True)
        delta = upd - vmean
        d0 = delta.astype(jnp.bfloat16)
        d1 = (delta - d0.astype(jnp.float32)).astype(jnp.bfloat16)
        o_ref[j] = vmean + jax.lax.dot_general(
            jnp.concatenate([ohs[j], ohs[j]], axis=0),
            jnp.concatenate([d0, d1], axis=0),
            (((0,), (0,)), ((), ())),
            preferred_element_type=jnp.float32,
        )


def kernel(queries, keys, values):
    b = queries.shape[0]
    counts_t = jnp.asarray(_COUNTS_T)
    context = pl.pallas_call(
        _body,
        grid=(b // _BB,),
        in_specs=[
            pl.BlockSpec((_L, _L), lambda i: (0, 0)),
            pl.BlockSpec((_BB, _L, _D), lambda i: (i, 0, 0)),
            pl.BlockSpec((_BB, _L, _D), lambda i: (i, 0, 0)),
            pl.BlockSpec((_BB, _L, _D), lambda i: (i, 0, 0)),
        ],
        out_specs=pl.BlockSpec((_BB, _L, _D), lambda i: (i, 0, 0)),
        out_shape=jax.ShapeDtypeStruct((b, _L, _D), jnp.float32),
    )(counts_t, queries, keys, values)
    return (context, None)


# speedup vs baseline: 4.2305x; 1.0310x over previous
import math

import jax
import jax.numpy as jnp
import numpy as np
from jax.experimental import pallas as pl

_L = 2048
_D = 64
_U = 40
_CHUNK = 256
_BB = 4
_NEG = -1e30


def _threefry2x32(k1, k2, x1, x2):
    rot = ([13, 15, 26, 6], [17, 29, 16, 24])
    k1 = np.uint32(k1)
    k2 = np.uint32(k2)
    ks = [k1, k2, np.uint32(k1 ^ k2 ^ np.uint32(0x1BD11BDA))]
    x = [x1.astype(np.uint32) + ks[0], x2.astype(np.uint32) + ks[1]]
    for i in range(5):
        for r in rot[i % 2]:
            x[0] = x[0] + x[1]
            x[1] = (x[1] << np.uint32(r)) | (x[1] >> np.uint32(32 - r))
            x[1] = x[0] ^ x[1]
        x[0] = x[0] + ks[(i + 1) % 3]
        x[1] = x[1] + ks[(i + 2) % 3] + np.uint32(i + 1)
    return x


def _sample_indices():
    size = _L * _U
    b1, b2 = _threefry2x32(
        0, 1234, np.zeros(2, np.uint32), np.arange(2, dtype=np.uint32)
    )
    h1, h2 = _threefry2x32(
        b1[1], b2[1],
        np.zeros(size, np.uint32), np.arange(size, dtype=np.uint32),
    )
    bits = (h1 ^ h2).reshape(_L, _U)
    return (bits % np.uint32(_L)).astype(np.int32)


def _build_tables():
    idx = _sample_indices()
    c = np.zeros((_L, _L), np.int8)
    np.add.at(c, (np.arange(_L)[:, None], idx), 1)
    return np.ascontiguousarray(c.T)


_COUNTS_T = _build_tables()


def _split3(x):
    x0 = x.astype(jnp.bfloat16)
    r1 = x - x0.astype(jnp.float32)
    x1 = r1.astype(jnp.bfloat16)
    x2 = (r1 - x1.astype(jnp.float32)).astype(jnp.bfloat16)
    return x0, x1, x2


def _body(ct_ref, q_ref, k_ref, v_ref, o_ref):
    qcats, kcats, vs = [], [], []
    for j in range(_BB):
        q0, q1, q2 = _split3(q_ref[j])
        k0, k1, k2 = _split3(k_ref[j])
        qcats.append(jnp.concatenate([q0, q0, q1, q0, q2, q1], axis=1))
        kcats.append(jnp.concatenate([k0, k1, k0, k2, k0, k1], axis=1))
        vs.append(v_ref[j])

    m_parts = [[] for _ in range(_BB)]
    for i in range(_L // _CHUNK):
        ct8 = ct_ref[:, i * _CHUNK:(i + 1) * _CHUNK]
        ct = ct8.astype(jnp.float32)
        mask = ct > 0.0
        for j in range(_BB):
            qcc = qcats[j][i * _CHUNK:(i + 1) * _CHUNK, :]
            st = jax.lax.dot_general(
                kcats[j], qcc, (((1,), (1,)), ((), ())),
                preferred_element_type=jnp.float32,
            )
            smax = jnp.max(jnp.where(mask, st, jnp.float32(_NEG)),
                           axis=0, keepdims=True)
            ssum = jnp.sum(st * ct, axis=0, keepdims=True)
            m_parts[j].append(smax - ssum * jnp.float32(1.0 / _L))
    ms = [jnp.concatenate(p, axis=1) for p in m_parts]

    lane_iota = jax.lax.broadcasted_iota(jnp.int32, (1, _L), 1)
    iota8 = (jax.lax.broadcasted_iota(jnp.int32, (16, 128), 0) * 128
             + jax.lax.broadcasted_iota(jnp.int32, (16, 128), 1))
    ms8 = [m.reshape(16, 128) for m in ms]
    oh_rows = [[] for _ in range(_BB)]
    for _ in range(_U):
        for j in range(_BB):
            gmax = jnp.max(ms8[j])
            idx = jnp.min(jnp.where(ms8[j] >= gmax, iota8, jnp.int32(_L)))
            oh_rows[j].append((lane_iota == idx).astype(jnp.bfloat16))
            ms8[j] = jnp.where(iota8 == idx, jnp.float32(_NEG), ms8[j])
    ohs = [jnp.concatenate(r, axis=0) for r in oh_rows]

    sc = []
    for j in range(_BB):
        q_r_cat = jax.lax.dot_general(
            ohs[j], qcats[j], (((1,), (0,)), ((), ())),
            preferred_element_type=jnp.float32,
        ).astype(jnp.bfloat16)
        sc.append(jax.lax.dot_general(
            q_r_cat, kcats[j], (((1,), (1,)), ((), ())),
            preferred_element_type=jnp.float32,
        ) * jnp.float32(1.0 / math.sqrt(_D)))
    scores = jnp.concatenate(sc, axis=0)
    smax = jnp.max(scores, axis=1, keepdims=True)
    e = jnp.exp(scores - smax)
    p = e / jnp.sum(e, axis=1, keepdims=True)
    p16 = p.astype(jnp.bfloat16)

    for j in range(_BB):
        v = vs[j]
        v0 = v.astype(jnp.bfloat16)
        v1 = (v - v0.astype(jnp.float32)).astype(jnp.bfloat16)
        upd = jax.lax.dot_general(
            p16[j * _U:(j + 1) * _U], jnp.concatenate([v0, v1], axis=1),
            (((1,), (0,)), ((), ())),
            preferred_element_type=jnp.float32,
        )
        upd = upd[:, :_D] + upd[:, _D:]
        vmean = jnp.mean(v, axis=0, keepdims=True)
        delta = upd - vmean
        d0 = delta.astype(jnp.bfloat16)
        d1 = (delta - d0.astype(jnp.float32)).astype(jnp.bfloat16)
        o_ref[j] = vmean + jax.lax.dot_general(
            jnp.concatenate([ohs[j], ohs[j]], axis=0),
            jnp.concatenate([d0, d1], axis=0),
            (((0,), (0,)), ((), ())),
            preferred_element_type=jnp.float32,
        )


def kernel(queries, keys, values):
    b = queries.shape[0]
    counts_t = jnp.asarray(_COUNTS_T)
    context = pl.pallas_call(
        _body,
        grid=(b // _BB,),
        in_specs=[
            pl.BlockSpec((_L, _L), lambda i: (0, 0)),
            pl.BlockSpec((_BB, _L, _D), lambda i: (i, 0, 0)),
            pl.BlockSpec((_BB, _L, _D), lambda i: (i, 0, 0)),
            pl.BlockSpec((_BB, _L, _D), lambda i: (i, 0, 0)),
        ],
        out_specs=pl.BlockSpec((_BB, _L, _D), lambda i: (i, 0, 0)),
        out_shape=jax.ShapeDtypeStruct((b, _L, _D), jnp.float32),
    )(counts_t, queries, keys, values)
    return (context, None)
